# Initial kernel scaffold; baseline (speedup 1.0000x reference)
import jax
import jax.numpy as jnp
from jax.experimental import pallas as pl


def kernel(features, codebook, Wt, bt, Wf, bf):
    raise NotImplementedError("write your pallas kernel here")


# speedup vs baseline: 1.2845x; 1.2845x over previous
import jax
import jax.numpy as jnp
from jax.experimental import pallas as pl
from jax.experimental.pallas import tpu as pltpu
from jax.experimental.pallas import tpu_sc as plsc


_M_BLK = 512
_GATHER_W = 128


def _argmin_body(flat_ref, wtT_ref, bt_ref, cbT_ref, idx_ref, c2_ref):
    @pl.when(pl.program_id(0) == 0)
    def _():
        cbT = cbT_ref[...]
        c2_ref[...] = jnp.sum(cbT * cbT, axis=0, keepdims=True)

    fc = jax.lax.dot_general(
        flat_ref[...], wtT_ref[...], (((1,), (0,)), ((), ())),
        preferred_element_type=jnp.float32) + bt_ref[...]
    a = jnp.sum(fc * fc, axis=1, keepdims=True)
    m = jax.lax.dot_general(
        fc, cbT_ref[...], (((1,), (0,)), ((), ())),
        preferred_element_type=jnp.float32)
    dist = a - 2.0 * m + c2_ref[...]
    bmin = jnp.min(dist, axis=1, keepdims=True)
    ii = jax.lax.broadcasted_iota(jnp.int32, dist.shape, 1)
    sentinel = jnp.int32(dist.shape[1])
    idx_ref[...] = jnp.min(
        jnp.where(dist == bmin, ii, sentinel), axis=1, keepdims=True)


def _out_body(flat_ref, q_ref, wfT_ref, bf_ref, o_ref):
    q16 = q_ref[...].astype(jnp.bfloat16)
    qo = jax.lax.dot_general(
        q16, wfT_ref[...], (((1,), (0,)), ((), ())),
        preferred_element_type=jnp.float32) + bf_ref[...]
    fl = flat_ref[...]
    o_ref[...] = fl + (qo - fl)


def _sc_gather(table, idx2d, n, d_code):
    mesh = plsc.VectorSubcoreMesh(core_axis_name="core", subcore_axis_name="subcore")

    @pl.kernel(out_type=jax.ShapeDtypeStruct((n, d_code), table.dtype), mesh=mesh)
    def gather_kernel(x_hbm, i_hbm, o_hbm):
        def body(i_vmem, o_vmem):
            pltpu.sync_copy(x_hbm.at[i_vmem.at[0]], o_vmem)

        pltpu.emit_pipeline(
            body,
            grid=(n // _GATHER_W,),
            in_specs=[pl.BlockSpec((1, _GATHER_W), index_map=lambda i: (0, i))],
            out_specs=[pl.BlockSpec((_GATHER_W, d_code), index_map=lambda i: (i, 0))],
            core_axis_name=("core", "subcore"),
            dimension_semantics=(pltpu.PARALLEL,),
        )(i_hbm, o_hbm)

    return gather_kernel(table, idx2d)


def kernel(features, codebook, Wt, bt, Wf, bf):
    orig_shape = features.shape
    d_in = orig_shape[-1]
    flat = features.reshape(-1, d_in)
    n = flat.shape[0]
    k, d_code = codebook.shape

    wtT = Wt.T
    cbT = codebook.T
    wfT = Wf.T
    bt2 = bt.reshape(1, d_code)
    bf2 = bf.reshape(1, d_in)

    nsteps = n // _M_BLK
    idx2d = pl.pallas_call(
        _argmin_body,
        grid=(nsteps,),
        in_specs=[
            pl.BlockSpec((_M_BLK, d_in), lambda i: (i, 0)),
            pl.BlockSpec((d_in, d_code), lambda i: (0, 0)),
            pl.BlockSpec((1, d_code), lambda i: (0, 0)),
            pl.BlockSpec((d_code, k), lambda i: (0, 0)),
        ],
        out_specs=pl.BlockSpec((_M_BLK, 1), lambda i: (i, 0)),
        out_shape=jax.ShapeDtypeStruct((n, 1), jnp.int32),
        scratch_shapes=[pltpu.VMEM((1, k), jnp.float32)],
        compiler_params=pltpu.CompilerParams(
            dimension_semantics=("arbitrary",)),
    )(flat, wtT, bt2, cbT)

    indices = idx2d[:, 0]
    quantized = _sc_gather(codebook, indices.reshape(1, n), n, d_code)

    out = pl.pallas_call(
        _out_body,
        grid=(n // 1024,),
        in_specs=[
            pl.BlockSpec((1024, d_in), lambda i: (i, 0)),
            pl.BlockSpec((1024, d_code), lambda i: (i, 0)),
            pl.BlockSpec((d_code, d_in), lambda i: (0, 0)),
            pl.BlockSpec((1, d_in), lambda i: (0, 0)),
        ],
        out_specs=pl.BlockSpec((1024, d_in), lambda i: (i, 0)),
        out_shape=jax.ShapeDtypeStruct((n, d_in), jnp.float32),
        compiler_params=pltpu.CompilerParams(
            dimension_semantics=("arbitrary",)),
    )(flat, quantized, wfT, bf2)

    return out.reshape(orig_shape), indices.reshape(orig_shape[:-1])
